# Initial kernel scaffold; baseline (speedup 1.0000x reference)
import jax
import jax.numpy as jnp
from jax.experimental import pallas as pl


def kernel(x, edge_index, W1, b1, W2, b2):
    raise NotImplementedError("write your pallas kernel here")


# speedup vs baseline: 9.8748x; 9.8748x over previous
import functools

import jax
import jax.numpy as jnp
from jax import lax
from jax.experimental import pallas as pl
from jax.experimental.pallas import tpu as pltpu
from jax.experimental.pallas import tpu_sc as plsc

N = 10000
E = 320000
D_IN = 128
D_HID = 128
D_OUT = 237

NPAD = 10240
NC, NS = 2, 16
NW = NC * NS
CH = 128
KPW = 79
EPAD = NW * KPW * CH
RPT = NPAD // NS

_mesh = plsc.VectorSubcoreMesh(core_axis_name="c", subcore_axis_name="s")


@functools.partial(
    pl.kernel,
    mesh=_mesh,
    out_type=jax.ShapeDtypeStruct((2 * NPAD, D_IN), jnp.float32),
    scratch_types=[
        pltpu.VMEM((KPW, CH), jnp.int32),
        pltpu.VMEM((KPW, CH), jnp.int32),
        pltpu.VMEM((CH, D_IN), jnp.float32),
        pltpu.VMEM_SHARED((NPAD, D_IN), jnp.float32),
        pltpu.SemaphoreType.DMA,
    ],
)
def _sc_aggregate(table, src_hbm, dst_hbm, zeros_hbm, out, src_v, dst_v,
                  rows_v, acc, sem):
    cid = lax.axis_index("c")
    sid = lax.axis_index("s")
    wid = sid * NC + cid
    pltpu.sync_copy(src_hbm.at[wid], src_v)
    pltpu.sync_copy(dst_hbm.at[wid], dst_v)
    pltpu.sync_copy(zeros_hbm.at[pl.ds(sid * RPT, RPT)],
                    acc.at[pl.ds(sid * RPT, RPT)])
    plsc.subcore_barrier()

    def body(j, c):
        pltpu.async_copy(table.at[src_v.at[j]], rows_v, sem).wait()
        pltpu.sync_copy(rows_v, acc.at[dst_v.at[j]], add=True)
        return c

    lax.fori_loop(0, KPW, body, 0)
    plsc.subcore_barrier()
    pltpu.sync_copy(acc.at[pl.ds(sid * RPT, RPT)],
                    out.at[pl.ds(cid * NPAD + sid * RPT, RPT)])


_DW = 16


@functools.partial(
    pl.kernel,
    mesh=_mesh,
    out_type=jax.ShapeDtypeStruct((2 * NPAD, _DW), jnp.float32),
    scratch_types=[
        pltpu.VMEM((KPW, CH), jnp.int32),
        pltpu.VMEM((CH, _DW), jnp.float32),
        pltpu.VMEM_SHARED((NPAD, _DW), jnp.float32),
    ],
)
def _sc_degree(dst_hbm, ones_hbm, zeros_hbm, out, dst_v, ones_v, acc):
    cid = lax.axis_index("c")
    sid = lax.axis_index("s")
    wid = sid * NC + cid
    pltpu.sync_copy(dst_hbm.at[wid], dst_v)
    pltpu.sync_copy(ones_hbm, ones_v)
    pltpu.sync_copy(zeros_hbm.at[pl.ds(sid * RPT, RPT)],
                    acc.at[pl.ds(sid * RPT, RPT)])
    plsc.subcore_barrier()

    def body(j, c):
        pltpu.sync_copy(ones_v, acc.at[dst_v.at[j]], add=True)
        return c

    lax.fori_loop(0, KPW, body, 0)
    plsc.subcore_barrier()
    pltpu.sync_copy(acc.at[pl.ds(sid * RPT, RPT)],
                    out.at[pl.ds(cid * NPAD + sid * RPT, RPT)])


_RB = 2560


def _scale_body(c0_ref, c1_ref, x_ref, xs_ref, dis_ref):
    deg = c0_ref[...] + c1_ref[...] + 1.0
    dis = lax.rsqrt(deg)
    dis_ref[...] = dis
    xs_ref[...] = x_ref[...] * dis


def _tc_scale(c0, c1, xp):
    grid = NPAD // _RB
    return pl.pallas_call(
        _scale_body,
        grid=(grid,),
        in_specs=[
            pl.BlockSpec((_RB, 1), lambda i: (i, 0)),
            pl.BlockSpec((_RB, 1), lambda i: (i, 0)),
            pl.BlockSpec((_RB, D_IN), lambda i: (i, 0)),
        ],
        out_specs=[
            pl.BlockSpec((_RB, D_IN), lambda i: (i, 0)),
            pl.BlockSpec((_RB, 1), lambda i: (i, 0)),
        ],
        out_shape=[
            jax.ShapeDtypeStruct((NPAD, D_IN), jnp.float32),
            jax.ShapeDtypeStruct((NPAD, 1), jnp.float32),
        ],
    )(c0, c1, xp)


def _mm1_body(a0_ref, a1_ref, xs_ref, dis_ref, w_ref, b_ref, o_ref):
    dis = dis_ref[...]
    t = dis * (a0_ref[...] + a1_ref[...] + xs_ref[...])
    h = jnp.dot(t, w_ref[...], preferred_element_type=jnp.float32)
    h = jnp.maximum(h + b_ref[...], 0.0)
    o_ref[...] = h * dis


def _tc_mm1(a0, a1, xs, dis, w, b):
    grid = NPAD // _RB
    return pl.pallas_call(
        _mm1_body,
        grid=(grid,),
        in_specs=[
            pl.BlockSpec((_RB, D_IN), lambda i: (i, 0)),
            pl.BlockSpec((_RB, D_IN), lambda i: (i, 0)),
            pl.BlockSpec((_RB, D_IN), lambda i: (i, 0)),
            pl.BlockSpec((_RB, 1), lambda i: (i, 0)),
            pl.BlockSpec((D_IN, D_HID), lambda i: (0, 0)),
            pl.BlockSpec((1, D_HID), lambda i: (0, 0)),
        ],
        out_specs=pl.BlockSpec((_RB, D_HID), lambda i: (i, 0)),
        out_shape=jax.ShapeDtypeStruct((NPAD, D_HID), jnp.float32),
    )(a0, a1, xs, dis, w, b)


_DOP = 256


def _mm2_body(a0_ref, a1_ref, hs_ref, dis_ref, w_ref, b_ref, o_ref):
    t = dis_ref[...] * (a0_ref[...] + a1_ref[...] + hs_ref[...])
    h = jnp.dot(t, w_ref[...], preferred_element_type=jnp.float32)
    o_ref[...] = h + b_ref[...]


def _tc_mm2(a0, a1, hs, dis, w, b):
    grid = NPAD // _RB
    return pl.pallas_call(
        _mm2_body,
        grid=(grid,),
        in_specs=[
            pl.BlockSpec((_RB, D_HID), lambda i: (i, 0)),
            pl.BlockSpec((_RB, D_HID), lambda i: (i, 0)),
            pl.BlockSpec((_RB, D_HID), lambda i: (i, 0)),
            pl.BlockSpec((_RB, 1), lambda i: (i, 0)),
            pl.BlockSpec((D_HID, _DOP), lambda i: (0, 0)),
            pl.BlockSpec((1, _DOP), lambda i: (0, 0)),
        ],
        out_specs=pl.BlockSpec((_RB, _DOP), lambda i: (i, 0)),
        out_shape=jax.ShapeDtypeStruct((NPAD, _DOP), jnp.float32),
    )(a0, a1, hs, dis, w, b)


def kernel(x, edge_index, W1, b1, W2, b2):
    src = edge_index[0].astype(jnp.int32)
    dst = edge_index[1].astype(jnp.int32)
    pad_idx = jnp.full((EPAD - E,), N, jnp.int32)
    src = jnp.concatenate([src, pad_idx]).reshape(NW, KPW, CH)
    dst = jnp.concatenate([dst, pad_idx]).reshape(NW, KPW, CH)

    xp = jnp.pad(x, ((0, NPAD - N), (0, 0)))
    zeros = jnp.zeros((NPAD, D_IN), jnp.float32)
    ones_table = jnp.ones((NPAD, D_IN), jnp.float32)

    cnt = _sc_aggregate(ones_table, dst, dst, zeros)
    xs, dis = _tc_scale(cnt[:NPAD, :1], cnt[NPAD:, :1], xp)

    agg1 = _sc_aggregate(xs, src, dst, zeros)
    h1s = _tc_mm1(agg1[:NPAD], agg1[NPAD:], xs, dis, W1,
                  b1.reshape(1, D_HID))

    agg2 = _sc_aggregate(h1s, src, dst, zeros)
    w2p = jnp.pad(W2, ((0, 0), (0, _DOP - D_OUT)))
    b2p = jnp.pad(b2, (0, _DOP - D_OUT)).reshape(1, _DOP)
    out = _tc_mm2(agg2[:NPAD], agg2[NPAD:], h1s, dis, w2p, b2p)
    return out[:N, :D_OUT]
